# baseline (speedup 1.0000x reference)
import numpy as np
from itertools import combinations

import jax
import jax.numpy as jnp
from jax import lax
from jax.experimental import pallas as pl
from jax.experimental.pallas import tpu as pltpu
from jax.experimental.pallas import tpu_sc as plsc

_SIZE_IN = 14
_HOTNESS = 3
_BATCH = 4096
_IN_DIM = 2 ** _SIZE_IN
_NSEL = 364
_NBAND = _BATCH // 8
_NT = _IN_DIM // 128
_D = 64

_FILT = np.array([sum(2 ** i for i in c) for c in combinations(range(_SIZE_IN), _HOTNESS)],
                 dtype=np.int32)
_G64 = np.unique(_FILT >> 6)
_NG = len(_G64)

_STAT = ((_G64 >> 1) * 16 + (_G64 & 1)).astype(np.int32)
_NENT = 8 * _NG
_NENTP = ((_NENT + 15) // 16) * 16
_SIDX = np.empty((_NENTP,), np.int32)
for _s in range(_NG):
    _SIDX[_s * 8:(_s + 1) * 8] = _STAT[_s] + 2 * np.arange(8, dtype=np.int32)
_SIDX[_NENT:] = _SIDX[_NENT - 1]

_SLOT = {int(g): i for i, g in enumerate(_G64)}
_SROW8 = np.array([8 * _SLOT[int(f) >> 6] for f in _FILT], dtype=np.int32)
_SCOL = (_FILT & 63).astype(np.int32)

_NVEC = 23
_SROWP = np.concatenate([_SROW8, np.full(4, _SROW8[-1], np.int32)])
_SCOLP = np.concatenate([_SCOL, np.full(4, _SCOL[-1], np.int32)])
_WROW = np.stack([_SROWP[16 * v:16 * v + 16] for v in range(_NVEC)]).astype(np.int32)
_WCOL = np.stack([_SCOLP[16 * v:16 * v + 16] for v in range(_NVEC)]).astype(np.int32)

_NCORES = 2
_NSUB = 16
_NTILES = _NCORES * _NSUB
_NCHUNK = _NBAND // _NTILES


def _body(tab_hbm, sidx_hbm, wrow_hbm, wcol_hbm, out_hbm,
          gbuf, idxbuf, sidx_v, wrow_v, wcol_v, obuf, sems):
    wid = lax.axis_index("s") * _NCORES + lax.axis_index("c")
    pltpu.sync_copy(sidx_hbm, sidx_v)
    pltpu.sync_copy(wrow_hbm, wrow_v)
    pltpu.sync_copy(wcol_hbm, wcol_v)

    def fire(c, buf):
        band = wid * _NCHUNK + c
        base = band * 2048
        iv = idxbuf.at[buf]
        for k in range(_NENTP // 16):
            iv[pl.ds(16 * k, 16)] = sidx_v[pl.ds(16 * k, 16)] + base
        pltpu.make_async_copy(tab_hbm.at[iv], gbuf.at[buf],
                              sems.at[buf]).start()

    def drain(buf):
        pltpu.make_async_copy(tab_hbm.at[pl.ds(0, _NENTP), :], gbuf.at[buf],
                              sems.at[buf]).wait()

    def compact(c, buf):
        gv = gbuf.at[buf]

        def row_body(r, _):
            for v in range(_NVEC):
                x = plsc.load_gather(gv, [wrow_v[v] + r, wcol_v[v]])
                obuf[16 * v // 128, r, pl.ds((16 * v) % 128, 16)] = x * 12.0 - 6.0
            return 0

        lax.fori_loop(0, 8, row_body, 0, unroll=False)
        band = wid * _NCHUNK + c
        pltpu.sync_copy(obuf, out_hbm.at[band])

    fire(0, 0)

    def chunk_body(c, _):
        buf = lax.rem(c, 2)

        @pl.when(c + 1 < _NCHUNK)
        def _():
            fire(c + 1, 1 - buf)

        drain(buf)
        compact(c, buf)
        return 0

    lax.fori_loop(0, _NCHUNK, chunk_body, 0, unroll=False)


def kernel(input_var):
    tab = (input_var.reshape(_NBAND, 8, _NT, 128)
           .transpose(0, 2, 1, 3)
           .reshape(_NBAND * _NT * 8 * 2, _D))
    sidx = jnp.asarray(_SIDX)
    wrow = jnp.asarray(_WROW)
    wcol = jnp.asarray(_WCOL)
    mesh = plsc.VectorSubcoreMesh(core_axis_name="c", subcore_axis_name="s",
                                  num_cores=_NCORES, num_subcores=_NSUB)
    out4 = pl.kernel(
        _body,
        out_type=jax.ShapeDtypeStruct((_NBAND, 3, 8, 128), jnp.float32),
        mesh=mesh,
        scratch_types=[
            pltpu.VMEM((2, _NENTP, _D), jnp.float32),
            pltpu.VMEM((2, _NENTP), jnp.int32),
            pltpu.VMEM((_NENTP,), jnp.int32),
            pltpu.VMEM((_NVEC, 16), jnp.int32),
            pltpu.VMEM((_NVEC, 16), jnp.int32),
            pltpu.VMEM((3, 8, 128), jnp.float32),
            pltpu.SemaphoreType.DMA((2,)),
        ],
        compiler_params=pltpu.CompilerParams(needs_layout_passes=False,
                                             use_tc_tiling_on_sc=False),
    )(tab, sidx, wrow, wcol)
    return out4.transpose(0, 2, 1, 3).reshape(_BATCH, 384)[:, :_NSEL]


# speedup vs baseline: 1.0321x; 1.0321x over previous
import numpy as np
from itertools import combinations

import jax
import jax.numpy as jnp
from jax import lax
from jax.experimental import pallas as pl
from jax.experimental.pallas import tpu as pltpu
from jax.experimental.pallas import tpu_sc as plsc

_SIZE_IN = 14
_HOTNESS = 3
_BATCH = 4096
_IN_DIM = 2 ** _SIZE_IN
_NSEL = 364
_NBAND = _BATCH // 8
_NT = _IN_DIM // 128
_D = 512

_FILT = np.array([sum(2 ** i for i in c) for c in combinations(range(_SIZE_IN), _HOTNESS)],
                 dtype=np.int32)
_TILES = np.unique(_FILT >> 7)
_NE = len(_TILES)

_SIDX = (_TILES.astype(np.int32) * 2)

_SLOT = {int(t): i for i, t in enumerate(_TILES)}
_SROW = np.array([_SLOT[int(f) >> 7] for f in _FILT], dtype=np.int32)
_SCOL = (_FILT & 127).astype(np.int32)

_NVEC = 23
_SROWP = np.concatenate([_SROW, np.full(4, _SROW[-1], np.int32)])
_SCOLP = np.concatenate([_SCOL, np.full(4, _SCOL[-1], np.int32)])
_WROW = np.stack([_SROWP[16 * v:16 * v + 16] for v in range(_NVEC)])
_WCOL = np.stack([_SCOLP[16 * v:16 * v + 16] for v in range(_NVEC)])

_NCORES = 2
_NSUB = 16
_NTILES = _NCORES * _NSUB
_NCHUNK = (_BATCH // _NTILES) // 4


def _body(tab_hbm, sidx_hbm, wrow_hbm, wcol_hbm, out_hbm,
          gbuf, idxbuf, sidx_v, wrow_v, wcol_v, obuf, sems):
    wid = lax.axis_index("s") * _NCORES + lax.axis_index("c")
    pltpu.sync_copy(sidx_hbm, sidx_v)
    pltpu.sync_copy(wrow_hbm, wrow_v)
    pltpu.sync_copy(wcol_hbm, wcol_v)

    def fire(c, buf):
        half = wid * _NCHUNK + c
        base = lax.div(half, 2) * 256 + lax.rem(half, 2)
        iv = idxbuf.at[buf]
        for k in range(_NE // 16):
            iv[pl.ds(16 * k, 16)] = sidx_v[pl.ds(16 * k, 16)] + base
        pltpu.make_async_copy(tab_hbm.at[iv], gbuf.at[buf],
                              sems.at[buf]).start()

    def drain(buf):
        pltpu.make_async_copy(tab_hbm.at[pl.ds(0, _NE), :], gbuf.at[buf],
                              sems.at[buf]).wait()

    def compact(c, buf):
        gv = gbuf.at[buf]

        def row_body(r, _):
            cbase = r * 128
            for v in range(_NVEC):
                x = plsc.load_gather(gv, [wrow_v[v], wcol_v[v] + cbase])
                obuf[16 * v // 128, r, pl.ds((16 * v) % 128, 16)] = x * 12.0 - 6.0
            return 0

        lax.fori_loop(0, 4, row_body, 0, unroll=False)
        half = wid * _NCHUNK + c
        band = lax.div(half, 2)
        p0 = lax.rem(half, 2) * 4
        pltpu.sync_copy(obuf, out_hbm.at[band, :, pl.ds(p0, 4), :])

    fire(0, 0)

    def chunk_body(c, _):
        buf = lax.rem(c, 2)

        @pl.when(c + 1 < _NCHUNK)
        def _():
            fire(c + 1, 1 - buf)

        drain(buf)
        compact(c, buf)
        return 0

    lax.fori_loop(0, _NCHUNK, chunk_body, 0, unroll=False)


def kernel(input_var):
    tab = (input_var.reshape(_NBAND, 8, _NT, 128)
           .transpose(0, 2, 1, 3)
           .reshape(_NBAND * _NT * 2, _D))
    sidx = jnp.asarray(_SIDX)
    wrow = jnp.asarray(_WROW.astype(np.int32))
    wcol = jnp.asarray(_WCOL.astype(np.int32))
    mesh = plsc.VectorSubcoreMesh(core_axis_name="c", subcore_axis_name="s",
                                  num_cores=_NCORES, num_subcores=_NSUB)
    out4 = pl.kernel(
        _body,
        out_type=jax.ShapeDtypeStruct((_NBAND, 3, 8, 128), jnp.float32),
        mesh=mesh,
        scratch_types=[
            pltpu.VMEM((2, _NE, _D), jnp.float32),
            pltpu.VMEM((2, _NE), jnp.int32),
            pltpu.VMEM((_NE,), jnp.int32),
            pltpu.VMEM((_NVEC, 16), jnp.int32),
            pltpu.VMEM((_NVEC, 16), jnp.int32),
            pltpu.VMEM((3, 4, 128), jnp.float32),
            pltpu.SemaphoreType.DMA((2,)),
        ],
        compiler_params=pltpu.CompilerParams(needs_layout_passes=False,
                                             use_tc_tiling_on_sc=False),
    )(tab, sidx, wrow, wcol)
    return out4.transpose(0, 2, 1, 3).reshape(_BATCH, 384)[:, :_NSEL]


# speedup vs baseline: 1.0574x; 1.0245x over previous
import numpy as np
from itertools import combinations

import jax
import jax.numpy as jnp
from jax import lax
from jax.experimental import pallas as pl
from jax.experimental.pallas import tpu as pltpu
from jax.experimental.pallas import tpu_sc as plsc

_SIZE_IN = 14
_HOTNESS = 3
_BATCH = 4096
_IN_DIM = 2 ** _SIZE_IN
_NSEL = 364
_NBAND = _BATCH // 8
_NT = _IN_DIM // 128
_D = 512

_FILT = np.array([sum(2 ** i for i in c) for c in combinations(range(_SIZE_IN), _HOTNESS)],
                 dtype=np.int32)
_TILES = np.unique(_FILT >> 7)
_NE = len(_TILES)

_SIDX = (_TILES.astype(np.int32) * 2)

_SLOT = {int(t): i for i, t in enumerate(_TILES)}
_SROW = np.array([_SLOT[int(f) >> 7] for f in _FILT], dtype=np.int32)
_SCOL = (_FILT & 127).astype(np.int32)

_NVEC = 23
_SROWP = np.concatenate([_SROW, np.full(4, _SROW[-1], np.int32)])
_SCOLP = np.concatenate([_SCOL, np.full(4, _SCOL[-1], np.int32)])
_WROW = np.stack([_SROWP[16 * v:16 * v + 16] for v in range(_NVEC)])
_WCOL = np.stack([_SCOLP[16 * v:16 * v + 16] for v in range(_NVEC)])

_PACKED = np.concatenate([
    _SIDX.reshape(4, 16),
    _WROW.astype(np.int32),
    _WCOL.astype(np.int32),
]).astype(np.int32)

_NCORES = 2
_NSUB = 16
_NTILES = _NCORES * _NSUB
_NCHUNK = (_BATCH // _NTILES) // 4


def _body(tab_hbm, tbl_hbm, out_hbm, gbuf, idxbuf, tbl_v, obuf, sems):
    wid = lax.axis_index("s") * _NCORES + lax.axis_index("c")
    pltpu.sync_copy(tbl_hbm, tbl_v)

    def fire(c, buf):
        half = wid * _NCHUNK + c
        base = lax.div(half, 2) * 256 + lax.rem(half, 2)
        iv = idxbuf.at[buf]
        for k in range(_NE // 16):
            iv[pl.ds(16 * k, 16)] = tbl_v[k] + base
        pltpu.make_async_copy(tab_hbm.at[iv], gbuf.at[buf],
                              sems.at[buf]).start()

    def drain(buf):
        pltpu.make_async_copy(tab_hbm.at[pl.ds(0, _NE), :], gbuf.at[buf],
                              sems.at[buf]).wait()

    def compact(c, buf):
        gv = gbuf.at[buf]

        def row_body(r, _):
            cbase = r * 128
            for v in range(_NVEC):
                x = plsc.load_gather(gv, [tbl_v[4 + v], tbl_v[27 + v] + cbase])
                obuf[16 * v // 128, r, pl.ds((16 * v) % 128, 16)] = x * 12.0 - 6.0
            return 0

        lax.fori_loop(0, 4, row_body, 0, unroll=False)
        half = wid * _NCHUNK + c
        band = lax.div(half, 2)
        p0 = lax.rem(half, 2) * 4
        pltpu.sync_copy(obuf, out_hbm.at[band, :, pl.ds(p0, 4), :])

    fire(0, 0)

    def chunk_body(c, _):
        buf = lax.rem(c, 2)

        @pl.when(c + 1 < _NCHUNK)
        def _():
            fire(c + 1, 1 - buf)

        drain(buf)
        compact(c, buf)
        return 0

    lax.fori_loop(0, _NCHUNK, chunk_body, 0, unroll=False)


def kernel(input_var):
    tab = (input_var.reshape(_NBAND, 8, _NT, 128)
           .transpose(0, 2, 1, 3)
           .reshape(_NBAND * _NT * 2, _D))
    tbl = jnp.asarray(_PACKED)
    mesh = plsc.VectorSubcoreMesh(core_axis_name="c", subcore_axis_name="s",
                                  num_cores=_NCORES, num_subcores=_NSUB)
    out4 = pl.kernel(
        _body,
        out_type=jax.ShapeDtypeStruct((_NBAND, 3, 8, 128), jnp.float32),
        mesh=mesh,
        scratch_types=[
            pltpu.VMEM((2, _NE, _D), jnp.float32),
            pltpu.VMEM((2, _NE), jnp.int32),
            pltpu.VMEM((50, 16), jnp.int32),
            pltpu.VMEM((3, 4, 128), jnp.float32),
            pltpu.SemaphoreType.DMA((2,)),
        ],
        compiler_params=pltpu.CompilerParams(needs_layout_passes=False,
                                             use_tc_tiling_on_sc=False),
    )(tab, tbl)
    return out4.transpose(0, 2, 1, 3).reshape(_BATCH, 384)[:, :_NSEL]
